# baseline (speedup 1.0000x reference)
import functools

import jax
import jax.numpy as jnp
from jax import lax
from jax.experimental import pallas as pl
from jax.experimental.pallas import tpu as pltpu
from jax.experimental.pallas import tpu_sc as plsc

_VOCAB = 1000
_EMBED = 128
_CONTEXT = 4
_BATCH = 16384
_HIDDEN = 128

_HALF = _BATCH // 2

_NC, _NS = 2, 16
_NW = _NC * _NS
_ROWS_PER_W = _HALF // _NW

_BB = 1024
_NBLK = _HALF // _BB


def _sc_gather(emb, idx):
    mesh = plsc.VectorSubcoreMesh(core_axis_name="c", subcore_axis_name="s")

    @functools.partial(
        pl.kernel,
        out_type=jax.ShapeDtypeStruct((_HALF, _EMBED), jnp.float32),
        mesh=mesh,
        scratch_types=[
            pltpu.VMEM((_ROWS_PER_W,), jnp.int32),
            pltpu.VMEM((_ROWS_PER_W, _EMBED), jnp.float32),
            pltpu.SemaphoreType.DMA,
        ],
    )
    def body(emb_hbm, idx_hbm, out_hbm, idx_v, rows_v, sem):
        wid = lax.axis_index("s") * _NC + lax.axis_index("c")
        base = wid * _ROWS_PER_W
        pltpu.sync_copy(idx_hbm.at[pl.ds(base, _ROWS_PER_W)], idx_v)
        pltpu.async_copy(emb_hbm.at[idx_v], rows_v, sem).wait()
        pltpu.sync_copy(rows_v, out_hbm.at[pl.ds(base, _ROWS_PER_W)])

    return body(emb, idx)


def _onehot_body(idx_ref, emb_ref, w1_ref, b1_ref, w2t_ref, b2_ref, out_ref):
    mt = lax.dot_general(w1_ref[...], emb_ref[...], (((0,), (1,)), ((), ())),
                         preferred_element_type=jnp.float32).astype(jnp.bfloat16)
    viota = lax.broadcasted_iota(jnp.int32, (_VOCAB, _BB), 0)
    diff = idx_ref[...] - viota
    onehot = (1 - jnp.minimum(jnp.abs(diff), 1)).astype(jnp.bfloat16)
    htp = jnp.dot(mt, onehot, preferred_element_type=jnp.float32)
    ht = jnp.maximum(htp + b1_ref[...], 0.0).astype(jnp.bfloat16)
    for c in range(_CONTEXT):
        out_ref[c] = (
            jnp.dot(w2t_ref[c], ht, preferred_element_type=jnp.float32)
            + b2_ref[c]
        )


def _tc_onehot(idx, emb, w1, b1, w2t, b2):
    return pl.pallas_call(
        _onehot_body,
        grid=(_NBLK,),
        in_specs=[
            pl.BlockSpec((1, _BB), lambda i: (0, i)),
            pl.BlockSpec((_VOCAB, _EMBED), lambda i: (0, 0)),
            pl.BlockSpec((_EMBED, _HIDDEN), lambda i: (0, 0)),
            pl.BlockSpec((_HIDDEN, 1), lambda i: (0, 0)),
            pl.BlockSpec((_CONTEXT, _VOCAB, _HIDDEN), lambda i: (0, 0, 0)),
            pl.BlockSpec((_CONTEXT, _VOCAB, 1), lambda i: (0, 0, 0)),
        ],
        out_specs=pl.BlockSpec((_CONTEXT, _VOCAB, _BB), lambda i: (0, 0, i)),
        out_shape=jax.ShapeDtypeStruct((_CONTEXT, _VOCAB, _BATCH), jnp.float32),
        compiler_params=pltpu.CompilerParams(
            dimension_semantics=("parallel",)),
    )(idx, emb, w1, b1, w2t, b2)


def _mlp_body(e_ref, w1_ref, b1_ref, w2t_ref, b2_ref, prev_ref, out_ref):
    del prev_ref
    ht = lax.dot_general(w1_ref[...], e_ref[...], (((0,), (1,)), ((), ())),
                         preferred_element_type=jnp.float32)
    ht = jnp.maximum(ht + b1_ref[...], 0.0).astype(jnp.bfloat16)
    for c in range(_CONTEXT):
        out_ref[c] = (
            jnp.dot(w2t_ref[c], ht, preferred_element_type=jnp.float32)
            + b2_ref[c]
        )


def _tc_mlp(e, w1, b1, w2t, b2, prev):
    return pl.pallas_call(
        _mlp_body,
        grid=(_NBLK,),
        in_specs=[
            pl.BlockSpec((_BB, _EMBED), lambda i: (i, 0)),
            pl.BlockSpec((_EMBED, _HIDDEN), lambda i: (0, 0)),
            pl.BlockSpec((_HIDDEN, 1), lambda i: (0, 0)),
            pl.BlockSpec((_CONTEXT, _VOCAB, _HIDDEN), lambda i: (0, 0, 0)),
            pl.BlockSpec((_CONTEXT, _VOCAB, 1), lambda i: (0, 0, 0)),
            pl.BlockSpec(memory_space=pl.ANY),
        ],
        out_specs=pl.BlockSpec((_CONTEXT, _VOCAB, _BB),
                               lambda i: (0, 0, i + _NBLK)),
        out_shape=jax.ShapeDtypeStruct((_CONTEXT, _VOCAB, _BATCH), jnp.float32),
        input_output_aliases={5: 0},
        compiler_params=pltpu.CompilerParams(
            dimension_semantics=("parallel",)),
    )(e, w1, b1, w2t, b2, prev)


def kernel(x, emb, W1, b1, W2, b2):
    xi = x.astype(jnp.int32)
    w2t = W2.T.reshape(_CONTEXT, _VOCAB, _HIDDEN).astype(jnp.bfloat16)
    b1c = b1.reshape(_HIDDEN, 1)
    b2c = b2.reshape(_CONTEXT, _VOCAB, 1)
    e1 = _sc_gather(emb, xi[_HALF:])
    out0 = _tc_onehot(xi[: _HALF].reshape(1, _HALF), emb, W1, b1c, w2t, b2c)
    out = _tc_mlp(e1, W1, b1c, w2t, b2c, out0)
    return out.transpose(2, 0, 1)


# speedup vs baseline: 1.0146x; 1.0146x over previous
import functools

import jax
import jax.numpy as jnp
from jax import lax
from jax.experimental import pallas as pl
from jax.experimental.pallas import tpu as pltpu
from jax.experimental.pallas import tpu_sc as plsc

_VOCAB = 1000
_EMBED = 128
_CONTEXT = 4
_BATCH = 16384
_HIDDEN = 128

_NC, _NS = 2, 16
_NW = _NC * _NS
_ROWS_PER_W = _BATCH // _NW


def _sc_gather(emb, idx):
    mesh = plsc.VectorSubcoreMesh(core_axis_name="c", subcore_axis_name="s")

    half = _ROWS_PER_W // 2

    @functools.partial(
        pl.kernel,
        out_type=jax.ShapeDtypeStruct((_BATCH, _EMBED), jnp.float32),
        mesh=mesh,
        scratch_types=[
            pltpu.VMEM((half,), jnp.int32),
            pltpu.VMEM((half,), jnp.int32),
            pltpu.VMEM((half, _EMBED), jnp.float32),
            pltpu.VMEM((half, _EMBED), jnp.float32),
            pltpu.SemaphoreType.DMA,
            pltpu.SemaphoreType.DMA,
            pltpu.SemaphoreType.DMA,
            pltpu.SemaphoreType.DMA,
        ],
    )
    def body(emb_hbm, idx_hbm, out_hbm,
             idx_v0, idx_v1, rows_v0, rows_v1, g0, g1, s0, s1):
        wid = lax.axis_index("s") * _NC + lax.axis_index("c")
        base = wid * _ROWS_PER_W
        pltpu.sync_copy(idx_hbm.at[pl.ds(base, half)], idx_v0)
        pltpu.sync_copy(idx_hbm.at[pl.ds(base + half, half)], idx_v1)
        cp0 = pltpu.async_copy(emb_hbm.at[idx_v0], rows_v0, g0)
        cp1 = pltpu.async_copy(emb_hbm.at[idx_v1], rows_v1, g1)
        cp0.wait()
        w0 = pltpu.async_copy(rows_v0, out_hbm.at[pl.ds(base, half)], s0)
        cp1.wait()
        w1 = pltpu.async_copy(rows_v1, out_hbm.at[pl.ds(base + half, half)], s1)
        w0.wait()
        w1.wait()

    return body(emb, idx)


_BB = 1024


def _mlp_body(e_ref, w1_ref, b1_ref, w2t_ref, b2_ref, out_ref):
    ht = lax.dot_general(w1_ref[...].astype(jnp.bfloat16),
                         e_ref[...].astype(jnp.bfloat16),
                         (((0,), (1,)), ((), ())),
                         preferred_element_type=jnp.float32)
    ht = jnp.maximum(ht + b1_ref[...], 0.0).astype(jnp.bfloat16)
    for c in range(_CONTEXT):
        out_ref[c] = (
            jnp.dot(w2t_ref[c], ht, preferred_element_type=jnp.float32)
            + b2_ref[c]
        )


def _tc_mlp(e, w1, b1, w2t, b2):
    grid = (_BATCH // _BB,)
    return pl.pallas_call(
        _mlp_body,
        grid=grid,
        in_specs=[
            pl.BlockSpec((_BB, _EMBED), lambda i: (i, 0)),
            pl.BlockSpec((_EMBED, _HIDDEN), lambda i: (0, 0)),
            pl.BlockSpec((_HIDDEN, 1), lambda i: (0, 0)),
            pl.BlockSpec((_CONTEXT, _VOCAB, _HIDDEN), lambda i: (0, 0, 0)),
            pl.BlockSpec((_CONTEXT, _VOCAB, 1), lambda i: (0, 0, 0)),
        ],
        out_specs=pl.BlockSpec((_CONTEXT, _VOCAB, _BB), lambda i: (0, 0, i)),
        out_shape=jax.ShapeDtypeStruct((_CONTEXT, _VOCAB, _BATCH), jnp.float32),
        compiler_params=pltpu.CompilerParams(
            dimension_semantics=("parallel",)),
    )(e, w1, b1, w2t, b2)


def kernel(x, emb, W1, b1, W2, b2):
    e = _sc_gather(emb, x.astype(jnp.int32))
    w2t = W2.T.reshape(_CONTEXT, _VOCAB, _HIDDEN).astype(jnp.bfloat16)
    out = _tc_mlp(e, W1, b1.reshape(_HIDDEN, 1), w2t,
                  b2.reshape(_CONTEXT, _VOCAB, 1))
    return out.transpose(2, 0, 1)
